# baseline (speedup 1.0000x reference)
import jax
import jax.numpy as jnp
from jax import lax
from jax.experimental import pallas as pl
from jax.experimental.pallas import tpu as pltpu
from jax.experimental.pallas import tpu_sc as plsc

B, C, H, L = 1024, 2, 50, 8
D = 64
N = B * C * H
NC, NS = 2, 16
NW = NC * NS
B_PER_W = B // NW
S_PER_W = N // NW
CS = 10
ROWS = CS * L
G = S_PER_W // CS
LANES = 16
SPB = C * H * L
SPC = H * L
SPB_S = C * H


def _body(x_hbm, tbl_hbm, emb_hbm, mask_hbm,
          idx4, idx_all, rows0, rows1, rows2, rows3,
          out0, out1, out2, out3, mask_all,
          gsem0, gsem1, gsem2, gsem3, osem0, osem1, osem2, osem3):
  wid = lax.axis_index("s") * NC + lax.axis_index("c")
  bbase = pl.multiple_of(wid * B_PER_W, B_PER_W)
  sbase = pl.multiple_of(wid * S_PER_W, S_PER_W)

  pltpu.sync_copy(x_hbm.at[pl.ds(bbase, B_PER_W)], idx4)

  iota = lax.iota(jnp.int32, LANES)
  hinc = iota // L
  lmod = iota - hinc * L
  zeros = jnp.zeros((LANES,), jnp.int32)

  def flatten(t, carry):
    for k in range(ROWS // LANES):
      base = t * ROWS + k * LANES
      b = base // SPB
      rem = base - b * SPB
      c = rem // SPC
      rem2 = rem - c * SPC
      h0 = rem2 // L
      v = plsc.load_gather(idx4, [zeros + b, zeros + c, hinc + h0, lmod])
      idx_all[t, pl.ds(k * LANES, LANES)] = v
    return carry

  lax.fori_loop(0, G, flatten, 0)

  rows_bufs = (rows0, rows1, rows2, rows3)
  out_bufs = (out0, out1, out2, out3)
  gsems = (gsem0, gsem1, gsem2, gsem3)
  osems = (osem0, osem1, osem2, osem3)
  NBUF = 4

  def start_gather(t, b):
    pltpu.async_copy(tbl_hbm.at[idx_all.at[t]], rows_bufs[b], gsems[b])

  for _t in range(NBUF):
    start_gather(_t, _t)

  def chunk(t, b):
    pltpu.make_async_copy(
        tbl_hbm.at[pl.ds(0, ROWS)], rows_bufs[b], gsems[b]).wait()

    s_vec = jnp.minimum(t * CS + iota, S_PER_W - 1)
    srow = s_vec // CS
    scol = (s_vec - srow * CS) * L
    cnt = jnp.zeros((LANES,), jnp.float32)
    for l in range(L):
      gi = plsc.load_gather(idx_all, [srow, scol + l])
      cnt = cnt + (gi != 0).astype(jnp.float32)
    recip = 1.0 / jnp.where(cnt == 0.0, 1.0, cnt)

    @pl.when(t >= NBUF)
    def _():
      pltpu.make_async_copy(
          out_bufs[b], emb_hbm.at[0, 0, pl.ds(0, CS), :], osems[b]).wait()

    rows = rows_bufs[b]
    out = out_bufs[b]
    for s in range(CS):
      r = recip[s]
      for c in range(D // LANES):
        col = pl.ds(c * LANES, LANES)
        acc = rows[s * L, col]
        for l in range(1, L):
          acc = acc + rows[s * L + l, col]
        out[s, col] = acc * r

    gs = t * CS
    ob = bbase + gs // SPB_S
    oc = (gs - (gs // SPB_S) * SPB_S) // H
    oh = gs - (gs // H) * H
    pltpu.async_copy(
        out, emb_hbm.at[ob, oc, pl.ds(oh, CS), :], osems[b])

    @pl.when(t + NBUF < G)
    def _():
      start_gather(t + NBUF, b)

  def body(i, carry):
    for b in range(NBUF):
      chunk(NBUF * i + b, b)
    return carry

  lax.fori_loop(0, G // NBUF, body, 0)

  def mask_pass(j, carry):
    s0 = j * LANES
    s_vec = s0 + iota
    srow = s_vec // CS
    scol = (s_vec - srow * CS) * L
    cnt = jnp.zeros((LANES,), jnp.float32)
    for l in range(L):
      gi = plsc.load_gather(idx_all, [srow, scol + l])
      cnt = cnt + (gi != 0).astype(jnp.float32)
    mask_all[pl.ds(s0, LANES)] = (cnt == 0.0).astype(jnp.float32)
    return carry

  lax.fori_loop(0, S_PER_W // LANES, mask_pass, 0)

  for _b in range(NBUF):
    pltpu.make_async_copy(
        out_bufs[_b], emb_hbm.at[0, 0, pl.ds(0, CS), :], osems[_b]).wait()
  pltpu.sync_copy(mask_all, mask_hbm.at[pl.ds(sbase, S_PER_W)])


_call = pl.kernel(
    _body,
    out_type=(
        jax.ShapeDtypeStruct((B, C, H, D), jnp.float32),
        jax.ShapeDtypeStruct((N,), jnp.float32),
    ),
    mesh=plsc.VectorSubcoreMesh(
        core_axis_name="c", subcore_axis_name="s",
        num_cores=NC, num_subcores=NS),
    compiler_params=pltpu.CompilerParams(
        needs_layout_passes=False, use_tc_tiling_on_sc=False),
    scratch_types=[
        pltpu.VMEM((B_PER_W, C, H, L), jnp.int32),
        pltpu.VMEM((G, ROWS), jnp.int32),
        pltpu.VMEM((ROWS, D), jnp.float32),
        pltpu.VMEM((ROWS, D), jnp.float32),
        pltpu.VMEM((ROWS, D), jnp.float32),
        pltpu.VMEM((ROWS, D), jnp.float32),
        pltpu.VMEM((CS, D), jnp.float32),
        pltpu.VMEM((CS, D), jnp.float32),
        pltpu.VMEM((CS, D), jnp.float32),
        pltpu.VMEM((CS, D), jnp.float32),
        pltpu.VMEM((S_PER_W,), jnp.float32),
        pltpu.SemaphoreType.DMA,
        pltpu.SemaphoreType.DMA,
        pltpu.SemaphoreType.DMA,
        pltpu.SemaphoreType.DMA,
        pltpu.SemaphoreType.DMA,
        pltpu.SemaphoreType.DMA,
        pltpu.SemaphoreType.DMA,
        pltpu.SemaphoreType.DMA,
    ],
)


def kernel(x, word_table):
  emb, mask = _call(x.astype(jnp.int32), word_table)
  return emb, mask.reshape(B, C, H)


# speedup vs baseline: 1.0136x; 1.0136x over previous
import jax
import jax.numpy as jnp
from jax import lax
from jax.experimental import pallas as pl
from jax.experimental.pallas import tpu as pltpu
from jax.experimental.pallas import tpu_sc as plsc

B, C, H, L = 1024, 2, 50, 8
D = 64
N = B * C * H
NC, NS = 2, 16
NW = NC * NS
B_PER_W = B // NW
S_PER_W = N // NW
CS = 10
ROWS = CS * L
G = S_PER_W // CS
LANES = 16
SPB = C * H * L
SPC = H * L
SPB_S = C * H


XR = N * L // 128
XR_PER_W = XR // NW


def _body(x_hbm, tbl_hbm, emb_hbm, mask_hbm,
          idx2, idx_all, rows0, rows1,
          out0, out1, mask_all,
          gsem0, gsem1, osem0, osem1):
  wid = lax.axis_index("s") * NC + lax.axis_index("c")
  bbase = pl.multiple_of(wid * B_PER_W, B_PER_W)
  sbase = pl.multiple_of(wid * S_PER_W, S_PER_W)

  pltpu.sync_copy(x_hbm.at[pl.ds(wid * XR_PER_W, XR_PER_W), :], idx2)

  iota = lax.iota(jnp.int32, LANES)
  zeros = jnp.zeros((LANES,), jnp.int32)

  def flatten(t, carry):
    for k in range(ROWS // LANES):
      base = t * ROWS + k * LANES
      row = base // 128
      col0 = base - row * 128
      v = plsc.load_gather(idx2, [zeros + row, col0 + iota])
      idx_all[t, pl.ds(k * LANES, LANES)] = v
    return carry

  lax.fori_loop(0, G, flatten, 0)

  rows_bufs = (rows0, rows1)
  out_bufs = (out0, out1)
  gsems = (gsem0, gsem1)
  osems = (osem0, osem1)
  NBUF = 2

  def start_gather(t, b):
    pltpu.async_copy(tbl_hbm.at[idx_all.at[t]], rows_bufs[b], gsems[b])

  for _t in range(NBUF):
    start_gather(_t, _t)

  def chunk(t, b):
    pltpu.make_async_copy(
        tbl_hbm.at[pl.ds(0, ROWS)], rows_bufs[b], gsems[b]).wait()

    s_vec = jnp.minimum(t * CS + iota, S_PER_W - 1)
    srow = s_vec // CS
    scol = (s_vec - srow * CS) * L
    cnt = jnp.zeros((LANES,), jnp.float32)
    for l in range(L):
      gi = plsc.load_gather(idx_all, [srow, scol + l])
      cnt = cnt + (gi != 0).astype(jnp.float32)
    recip = 1.0 / jnp.where(cnt == 0.0, 1.0, cnt)

    @pl.when(t >= NBUF)
    def _():
      pltpu.make_async_copy(
          out_bufs[b], emb_hbm.at[0, 0, pl.ds(0, CS), :], osems[b]).wait()

    rows = rows_bufs[b]
    out = out_bufs[b]
    for s in range(CS):
      r = recip[s]
      for c in range(D // LANES):
        col = pl.ds(c * LANES, LANES)
        acc = rows[s * L, col]
        for l in range(1, L):
          acc = acc + rows[s * L + l, col]
        out[s, col] = acc * r

    gs = t * CS
    ob = bbase + gs // SPB_S
    oc = (gs - (gs // SPB_S) * SPB_S) // H
    oh = gs - (gs // H) * H
    pltpu.async_copy(
        out, emb_hbm.at[ob, oc, pl.ds(oh, CS), :], osems[b])

    @pl.when(t + NBUF < G)
    def _():
      start_gather(t + NBUF, b)

  def body(i, carry):
    for b in range(NBUF):
      chunk(NBUF * i + b, b)
    return carry

  lax.fori_loop(0, G // NBUF, body, 0)

  def mask_pass(j, carry):
    s0 = j * LANES
    s_vec = s0 + iota
    srow = s_vec // CS
    scol = (s_vec - srow * CS) * L
    cnt = jnp.zeros((LANES,), jnp.float32)
    for l in range(L):
      gi = plsc.load_gather(idx_all, [srow, scol + l])
      cnt = cnt + (gi != 0).astype(jnp.float32)
    mask_all[pl.ds(s0, LANES)] = (cnt == 0.0).astype(jnp.float32)
    return carry

  lax.fori_loop(0, S_PER_W // LANES, mask_pass, 0)

  for _b in range(NBUF):
    pltpu.make_async_copy(
        out_bufs[_b], emb_hbm.at[0, 0, pl.ds(0, CS), :], osems[_b]).wait()
  pltpu.sync_copy(mask_all, mask_hbm.at[pl.ds(sbase, S_PER_W)])


_call = pl.kernel(
    _body,
    out_type=(
        jax.ShapeDtypeStruct((B, C, H, D), jnp.float32),
        jax.ShapeDtypeStruct((N,), jnp.float32),
    ),
    mesh=plsc.VectorSubcoreMesh(
        core_axis_name="c", subcore_axis_name="s",
        num_cores=NC, num_subcores=NS),
    compiler_params=pltpu.CompilerParams(
        needs_layout_passes=False, use_tc_tiling_on_sc=False),
    scratch_types=[
        pltpu.VMEM((XR_PER_W, 128), jnp.int32),
        pltpu.VMEM((G, ROWS), jnp.int32),
        pltpu.VMEM((ROWS, D), jnp.float32),
        pltpu.VMEM((ROWS, D), jnp.float32),
        pltpu.VMEM((CS, D), jnp.float32),
        pltpu.VMEM((CS, D), jnp.float32),
        pltpu.VMEM((S_PER_W,), jnp.float32),
        pltpu.SemaphoreType.DMA,
        pltpu.SemaphoreType.DMA,
        pltpu.SemaphoreType.DMA,
        pltpu.SemaphoreType.DMA,
    ],
)


def kernel(x, word_table):
  x2 = x.astype(jnp.int32).reshape(XR, 128)
  emb, mask = _call(x2, word_table)
  return emb, mask.reshape(B, C, H)


# speedup vs baseline: 1.1932x; 1.1773x over previous
import jax
import jax.numpy as jnp
from jax import lax
from jax.experimental import pallas as pl
from jax.experimental.pallas import tpu as pltpu
from jax.experimental.pallas import tpu_sc as plsc

B, C, H, L = 1024, 2, 50, 8
D = 64
N = B * C * H
NC, NS = 2, 16
NW = NC * NS
B_PER_W = B // NW
S_PER_W = N // NW
CS = 10
ROWS = CS * L
G = S_PER_W // CS
LANES = 16
SPB = C * H * L
SPC = H * L
SPB_S = C * H


def _body(x_hbm, tbl_hbm, emb_hbm, mask_hbm,
          idx5, idx_all, rows0, rows1,
          out0, out1, mask_all,
          gsem0, gsem1, osem0, osem1):
  wid = lax.axis_index("s") * NC + lax.axis_index("c")
  bbase = pl.multiple_of(wid * B_PER_W, B_PER_W)
  sbase = pl.multiple_of(wid * S_PER_W, S_PER_W)

  bt = wid // (128 // B_PER_W)
  bl0 = (wid % (128 // B_PER_W)) * B_PER_W
  pltpu.sync_copy(x_hbm.at[:, :, bt, :, pl.ds(bl0, B_PER_W)], idx5)

  iota = lax.iota(jnp.int32, LANES)
  hinc = iota // L
  lmod = iota - hinc * L
  zeros = jnp.zeros((LANES,), jnp.int32)

  def flatten(t, carry):
    for k in range(ROWS // LANES):
      base = t * ROWS + k * LANES
      b = base // SPB
      rem = base - b * SPB
      c = rem // SPC
      rem2 = rem - c * SPC
      h0 = rem2 // L
      v = plsc.load_gather(idx5, [zeros + c, hinc + h0, lmod, zeros + b])
      idx_all[t, pl.ds(k * LANES, LANES)] = v
    return carry

  lax.fori_loop(0, G, flatten, 0)

  rows_bufs = (rows0, rows1)
  out_bufs = (out0, out1)
  gsems = (gsem0, gsem1)
  osems = (osem0, osem1)
  NBUF = 2

  def start_gather(t, b):
    pltpu.async_copy(tbl_hbm.at[idx_all.at[t]], rows_bufs[b], gsems[b])

  for _t in range(NBUF):
    start_gather(_t, _t)

  def chunk(t, b):
    pltpu.make_async_copy(
        tbl_hbm.at[pl.ds(0, ROWS)], rows_bufs[b], gsems[b]).wait()

    s_vec = jnp.minimum(t * CS + iota, S_PER_W - 1)
    srow = s_vec // CS
    scol = (s_vec - srow * CS) * L
    cnt = jnp.zeros((LANES,), jnp.float32)
    for l in range(L):
      gi = plsc.load_gather(idx_all, [srow, scol + l])
      cnt = cnt + (gi != 0).astype(jnp.float32)
    recip = 1.0 / jnp.where(cnt == 0.0, 1.0, cnt)

    @pl.when(t >= NBUF)
    def _():
      pltpu.make_async_copy(
          out_bufs[b], emb_hbm.at[0, 0, pl.ds(0, CS), :], osems[b]).wait()

    rows = rows_bufs[b]
    out = out_bufs[b]
    for s in range(CS):
      r = recip[s]
      for c in range(D // LANES):
        col = pl.ds(c * LANES, LANES)
        acc = rows[s * L, col]
        for l in range(1, L):
          acc = acc + rows[s * L + l, col]
        out[s, col] = acc * r

    gs = t * CS
    ob = bbase + gs // SPB_S
    oc = (gs - (gs // SPB_S) * SPB_S) // H
    oh = gs - (gs // H) * H
    pltpu.async_copy(
        out, emb_hbm.at[ob, oc, pl.ds(oh, CS), :], osems[b])

    @pl.when(t + NBUF < G)
    def _():
      start_gather(t + NBUF, b)

  def body(i, carry):
    for b in range(NBUF):
      chunk(NBUF * i + b, b)
    return carry

  lax.fori_loop(0, G // NBUF, body, 0)

  def mask_pass(j, carry):
    s0 = j * LANES
    s_vec = s0 + iota
    srow = s_vec // CS
    scol = (s_vec - srow * CS) * L
    cnt = jnp.zeros((LANES,), jnp.float32)
    for l in range(L):
      gi = plsc.load_gather(idx_all, [srow, scol + l])
      cnt = cnt + (gi != 0).astype(jnp.float32)
    mask_all[pl.ds(s0, LANES)] = (cnt == 0.0).astype(jnp.float32)
    return carry

  lax.fori_loop(0, S_PER_W // LANES, mask_pass, 0)

  for _b in range(NBUF):
    pltpu.make_async_copy(
        out_bufs[_b], emb_hbm.at[0, 0, pl.ds(0, CS), :], osems[_b]).wait()
  pltpu.sync_copy(mask_all, mask_hbm.at[pl.ds(sbase, S_PER_W)])


_call = pl.kernel(
    _body,
    out_type=(
        jax.ShapeDtypeStruct((B, C, H, D), jnp.float32),
        jax.ShapeDtypeStruct((N,), jnp.float32),
    ),
    mesh=plsc.VectorSubcoreMesh(
        core_axis_name="c", subcore_axis_name="s",
        num_cores=NC, num_subcores=NS),
    compiler_params=pltpu.CompilerParams(
        needs_layout_passes=False, use_tc_tiling_on_sc=False),
    scratch_types=[
        pltpu.VMEM((C, H, L, B_PER_W), jnp.int32),
        pltpu.VMEM((G, ROWS), jnp.int32),
        pltpu.VMEM((ROWS, D), jnp.float32),
        pltpu.VMEM((ROWS, D), jnp.float32),
        pltpu.VMEM((CS, D), jnp.float32),
        pltpu.VMEM((CS, D), jnp.float32),
        pltpu.VMEM((S_PER_W,), jnp.float32),
        pltpu.SemaphoreType.DMA,
        pltpu.SemaphoreType.DMA,
        pltpu.SemaphoreType.DMA,
        pltpu.SemaphoreType.DMA,
    ],
)


def kernel(x, word_table):
  x5 = x.astype(jnp.int32).reshape(8, 128, C, H, L).transpose(2, 3, 0, 4, 1)
  emb, mask = _call(x5, word_table)
  return emb, mask.reshape(B, C, H)
